# baseline (speedup 1.0000x reference)
import functools

import jax
import jax.numpy as jnp
from jax import lax
from jax.experimental import pallas as pl
from jax.experimental.pallas import tpu as pltpu
from jax.experimental.pallas import tpu_sc as plsc

H = 224
W = 224
C = 96
B = 8
NPIX = H * W
NPLANES = B * C
NLANES = 16
NWORKERS = 32
WORKERS_PER_IMAGE = NWORKERS // B
CH_PER_WORKER = C // WORKERS_PER_IMAGE
CHUNKS_PER_ROW = W // NLANES
BLK_ROWS = 16
NBLK = H // BLK_ROWS
NBUF = 4
BLK_PX = BLK_ROWS * W
FMAX = NPIX - W - 2
P_UNROLL = 8
PLANE_PAD = W + 2 * NLANES


def _splat(vec, lane):
    return jnp.full((NLANES,), vec[lane], dtype=jnp.float32)


def _bf16_round(v):
    u = plsc.bitcast(v, jnp.uint32)
    r = (u + jnp.uint32(0x7FFF) + ((u >> jnp.uint32(16)) & jnp.uint32(1))) \
        & jnp.uint32(0xFFFF0000)
    return plsc.bitcast(r, jnp.float32)


def _affine_body(x_hbm, t_hbm, out_hbm, tv, f_v, plane_v,
                 ob0, ob1, ob2, ob3, sem0, sem1, sem2, sem3):
    cid = lax.axis_index("c")
    sid = lax.axis_index("s")
    wid = sid * 2 + cid
    b = wid // WORKERS_PER_IMAGE
    sub = wid % WORKERS_PER_IMAGE

    pltpu.sync_copy(t_hbm.at[b], tv)
    tvec = tv[...]

    i00 = _bf16_round(_splat(tvec, 0))
    i01 = _bf16_round(_splat(tvec, 1))
    i10 = _bf16_round(_splat(tvec, 2))
    i11 = _bf16_round(_splat(tvec, 3))
    ntx = _bf16_round(-_splat(tvec, 4))
    nty = _bf16_round(-_splat(tvec, 5))
    tpx = ntx * i00 + nty * i10
    tpy = ntx * i01 + nty * i11

    lim = jnp.float32(H - 2)
    zero = jnp.float32(0.0)
    cx0 = jnp.clip(tpx, zero, lim)
    cy0 = jnp.clip(tpy, zero, lim)
    dx0 = cx0 - cx0.astype(jnp.int32).astype(jnp.float32)
    dy0 = cy0 - cy0.astype(jnp.int32).astype(jnp.float32)
    w00 = (1.0 - dx0) * (1.0 - dy0)
    w10 = dx0 * (1.0 - dy0)
    w01 = (1.0 - dx0) * dy0
    w11 = dx0 * dy0

    lanes_f = lax.iota(jnp.int32, NLANES).astype(jnp.float32)

    def idx_row(y, carry):
        yf = jnp.full((NLANES,), y, dtype=jnp.int32).astype(jnp.float32)
        js = range(CHUNKS_PER_ROW)
        xfs = [lanes_f + jnp.float32(j * NLANES) for j in js]
        cxs = [jnp.clip((xfs[j] * i00 + yf * i10) + tpx, zero, lim) for j in js]
        cys = [jnp.clip((xfs[j] * i01 + yf * i11) + tpy, zero, lim) for j in js]
        fs = [jnp.clip(cys[j].astype(jnp.int32) * W + cxs[j].astype(jnp.int32),
                       0, FMAX) for j in js]
        for j in js:
            f_v[pl.ds(y * W + j * NLANES, NLANES)] = fs[j]
        return carry

    lax.fori_loop(0, H, idx_row, 0)

    out_bufs = (ob0, ob1, ob2, ob3)
    sems = (sem0, sem1, sem2, sem3)

    def chan_body(k, carry):
        plane = b * C + sub * CH_PER_WORKER + k
        pltpu.sync_copy(x_hbm.at[pl.ds(plane * NPIX, NPIX)],
                        plane_v.at[pl.ds(0, NPIX)])

        copies = [None] * NBUF
        for blk in range(NBLK):
            p = blk % NBUF
            buf = out_bufs[p]
            if copies[p] is not None:
                copies[p].wait()

            def blk_row(r, carry2):
                o = (blk * BLK_ROWS + r) * W
                fs = [f_v[pl.ds(o + j * NLANES, NLANES)]
                      for j in range(CHUNKS_PER_ROW)]
                gs = [plsc.load_gather(plane_v, [fs[j]])
                      for j in range(CHUNKS_PER_ROW)]
                for j in range(CHUNKS_PER_ROW):
                    buf[pl.ds(r * W + j * NLANES, NLANES)] = gs[j]
                return carry2

            lax.fori_loop(0, BLK_ROWS, blk_row, 0)
            copies[p] = pltpu.async_copy(
                buf,
                out_hbm.at[pl.ds(plane * NPIX + blk * BLK_PX, BLK_PX)],
                sems[p])
        for cp in copies:
            cp.wait()
        return carry

    lax.fori_loop(0, CH_PER_WORKER, chan_body, 0)


_affine_sc = functools.partial(
    pl.kernel,
    mesh=plsc.VectorSubcoreMesh(core_axis_name="c", subcore_axis_name="s"),
    out_type=jax.ShapeDtypeStruct((NPLANES * NPIX,), jnp.float32),
    compiler_params=pltpu.CompilerParams(needs_layout_passes=False),
    scratch_types=[
        pltpu.VMEM((NLANES,), jnp.float32),
        pltpu.VMEM((NPIX,), jnp.int32),
        pltpu.VMEM((NPIX + PLANE_PAD,), jnp.float32),
        pltpu.VMEM((BLK_PX,), jnp.float32),
        pltpu.VMEM((BLK_PX,), jnp.float32),
        pltpu.VMEM((BLK_PX,), jnp.float32),
        pltpu.VMEM((BLK_PX,), jnp.float32),
        pltpu.SemaphoreType.DMA,
        pltpu.SemaphoreType.DMA,
        pltpu.SemaphoreType.DMA,
        pltpu.SemaphoreType.DMA,
    ],
)(_affine_body)


def _fold_tc_body(w_ref, x_ref, o_ref):
    xv = x_ref[0]
    w00 = w_ref[0, 0, 0]
    w10 = w_ref[0, 0, 1]
    w01 = w_ref[0, 0, 2]
    w11 = w_ref[0, 0, 3]
    sx = jnp.concatenate([xv[:, 1:], xv[:, :1]], axis=1)
    sy = jnp.concatenate([xv[1:, :], xv[:1, :]], axis=0)
    sxy = jnp.concatenate([sy[:, 1:], sy[:, :1]], axis=1)
    o_ref[0] = w00 * xv + w10 * sx + w01 * sy + w11 * sxy


_fold_tc = pl.pallas_call(
    _fold_tc_body,
    grid=(NPLANES,),
    in_specs=[
        pl.BlockSpec((1, 1, 4), lambda i: (i, 0, 0), memory_space=pltpu.SMEM),
        pl.BlockSpec((1, H, W), lambda i: (i, 0, 0)),
    ],
    out_specs=pl.BlockSpec((1, H, W), lambda i: (i, 0, 0)),
    out_shape=jax.ShapeDtypeStruct((NPLANES, H, W), jnp.float32),
)


def _bf16_round_host(v):
    u = jax.lax.bitcast_convert_type(v, jnp.uint32)
    r = (u + jnp.uint32(0x7FFF) + ((u >> jnp.uint32(16)) & jnp.uint32(1))) \
        & jnp.uint32(0xFFFF0000)
    return jax.lax.bitcast_convert_type(r, jnp.float32)


@jax.jit
def kernel(x, transform):
    ainv = jnp.linalg.inv(transform[:, :4].reshape(B, 2, 2))
    params = jnp.concatenate([ainv.reshape(B, 4), transform[:, 4:6]], axis=1)
    params = jnp.pad(params, ((0, 0), (0, NLANES - 6)))

    i00 = _bf16_round_host(ainv[:, 0, 0])
    i01 = _bf16_round_host(ainv[:, 0, 1])
    i10 = _bf16_round_host(ainv[:, 1, 0])
    i11 = _bf16_round_host(ainv[:, 1, 1])
    ntx = _bf16_round_host(-transform[:, 4])
    nty = _bf16_round_host(-transform[:, 5])
    tpx = ntx * i00 + nty * i10
    tpy = ntx * i01 + nty * i11
    lim = jnp.float32(H - 2)
    cx0 = jnp.clip(tpx, 0.0, lim)
    cy0 = jnp.clip(tpy, 0.0, lim)
    dx0 = cx0 - cx0.astype(jnp.int32).astype(jnp.float32)
    dy0 = cy0 - cy0.astype(jnp.int32).astype(jnp.float32)
    w = jnp.stack([(1.0 - dx0) * (1.0 - dy0), dx0 * (1.0 - dy0),
                   (1.0 - dx0) * dy0, dx0 * dy0], axis=1)
    w_planes = jnp.repeat(w, C, axis=0).reshape(NPLANES, 1, 4)

    p_arr = _fold_tc(w_planes, x.reshape(NPLANES, H, W))
    out = _affine_sc(p_arr.reshape(NPLANES * NPIX), params)
    return out.reshape(x.shape)


# speedup vs baseline: 1.2453x; 1.2453x over previous
import functools

import jax
import jax.numpy as jnp
from jax import lax
from jax.experimental import pallas as pl
from jax.experimental.pallas import tpu as pltpu
from jax.experimental.pallas import tpu_sc as plsc

H = 224
W = 224
C = 96
B = 8
NPIX = H * W
NPLANES = B * C
NLANES = 16
NWORKERS = 32
WORKERS_PER_IMAGE = NWORKERS // B
CH_PER_WORKER = C // WORKERS_PER_IMAGE
CHUNKS_PER_ROW = W // NLANES
BLK_ROWS = 16
NBLK = H // BLK_ROWS
NBUF = 4
BLK_PX = BLK_ROWS * W
FMAX = NPIX - W - 2
P_UNROLL = 8
PLANE_PAD = W + 2 * NLANES


def _splat(vec, lane):
    return jnp.full((NLANES,), vec[lane], dtype=jnp.float32)


def _bf16_round(v):
    u = plsc.bitcast(v, jnp.uint32)
    r = (u + jnp.uint32(0x7FFF) + ((u >> jnp.uint32(16)) & jnp.uint32(1))) \
        & jnp.uint32(0xFFFF0000)
    return plsc.bitcast(r, jnp.float32)


def _affine_body(x_hbm, t_hbm, out_hbm, tv, f_v, plane_v,
                 ob0, ob1, ob2, ob3, sem0, sem1, sem2, sem3):
    cid = lax.axis_index("c")
    sid = lax.axis_index("s")
    wid = sid * 2 + cid
    b = wid // WORKERS_PER_IMAGE
    sub = wid % WORKERS_PER_IMAGE

    pltpu.sync_copy(t_hbm.at[b], tv)
    tvec = tv[...]

    i00 = _bf16_round(_splat(tvec, 0))
    i01 = _bf16_round(_splat(tvec, 1))
    i10 = _bf16_round(_splat(tvec, 2))
    i11 = _bf16_round(_splat(tvec, 3))
    ntx = _bf16_round(-_splat(tvec, 4))
    nty = _bf16_round(-_splat(tvec, 5))
    tpx = ntx * i00 + nty * i10
    tpy = ntx * i01 + nty * i11

    lim = jnp.float32(H - 2)
    zero = jnp.float32(0.0)
    cx0 = jnp.clip(tpx, zero, lim)
    cy0 = jnp.clip(tpy, zero, lim)
    dx0 = cx0 - cx0.astype(jnp.int32).astype(jnp.float32)
    dy0 = cy0 - cy0.astype(jnp.int32).astype(jnp.float32)
    w00 = (1.0 - dx0) * (1.0 - dy0)
    w10 = dx0 * (1.0 - dy0)
    w01 = (1.0 - dx0) * dy0
    w11 = dx0 * dy0

    lanes_f = lax.iota(jnp.int32, NLANES).astype(jnp.float32)

    def idx_row(y, carry):
        yf = jnp.full((NLANES,), y, dtype=jnp.int32).astype(jnp.float32)
        js = range(CHUNKS_PER_ROW)
        xfs = [lanes_f + jnp.float32(j * NLANES) for j in js]
        cxs = [jnp.clip((xfs[j] * i00 + yf * i10) + tpx, zero, lim) for j in js]
        cys = [jnp.clip((xfs[j] * i01 + yf * i11) + tpy, zero, lim) for j in js]
        fs = [jnp.clip(cys[j].astype(jnp.int32) * W + cxs[j].astype(jnp.int32),
                       0, FMAX) for j in js]
        for j in js:
            f_v[pl.ds(y * W + j * NLANES, NLANES)] = fs[j]
        return carry

    lax.fori_loop(0, H, idx_row, 0)

    out_bufs = (ob0, ob1, ob2, ob3)
    sems = (sem0, sem1, sem2, sem3)

    def chan_body(k, carry):
        plane = b * C + sub * CH_PER_WORKER + k
        pltpu.sync_copy(x_hbm.at[pl.ds(plane * NPIX, NPIX)],
                        plane_v.at[pl.ds(0, NPIX)])

        def p_body(m, carry2):
            bases = [(m * P_UNROLL + u) * NLANES for u in range(P_UNROLL)]
            a = [plane_v[pl.ds(o2, NLANES)] for o2 in bases]
            bb = [plane_v[pl.ds(o2 + 1, NLANES)] for o2 in bases]
            cc = [plane_v[pl.ds(o2 + W, NLANES)] for o2 in bases]
            dd = [plane_v[pl.ds(o2 + W + 1, NLANES)] for o2 in bases]
            vals = [w00 * a[u] + w10 * bb[u] + w01 * cc[u] + w11 * dd[u]
                    for u in range(P_UNROLL)]
            for u in range(P_UNROLL):
                plane_v[pl.ds(bases[u], NLANES)] = vals[u]
            return carry2

        lax.fori_loop(0, NPIX // (NLANES * P_UNROLL), p_body, 0)

        copies = [None] * NBUF
        for blk in range(NBLK):
            p = blk % NBUF
            buf = out_bufs[p]
            if copies[p] is not None:
                copies[p].wait()

            def blk_row(r, carry2):
                o = (blk * BLK_ROWS + r) * W
                fs = [f_v[pl.ds(o + j * NLANES, NLANES)]
                      for j in range(CHUNKS_PER_ROW)]
                gs = [plsc.load_gather(plane_v, [fs[j]])
                      for j in range(CHUNKS_PER_ROW)]
                for j in range(CHUNKS_PER_ROW):
                    buf[pl.ds(r * W + j * NLANES, NLANES)] = gs[j]
                return carry2

            lax.fori_loop(0, BLK_ROWS, blk_row, 0)
            copies[p] = pltpu.async_copy(
                buf,
                out_hbm.at[pl.ds(plane * NPIX + blk * BLK_PX, BLK_PX)],
                sems[p])
        for cp in copies:
            cp.wait()
        return carry

    lax.fori_loop(0, CH_PER_WORKER, chan_body, 0)


_affine_sc = functools.partial(
    pl.kernel,
    mesh=plsc.VectorSubcoreMesh(core_axis_name="c", subcore_axis_name="s"),
    out_type=jax.ShapeDtypeStruct((NPLANES * NPIX,), jnp.float32),
    compiler_params=pltpu.CompilerParams(needs_layout_passes=False),
    scratch_types=[
        pltpu.VMEM((NLANES,), jnp.float32),
        pltpu.VMEM((NPIX,), jnp.int32),
        pltpu.VMEM((NPIX + PLANE_PAD,), jnp.float32),
        pltpu.VMEM((BLK_PX,), jnp.float32),
        pltpu.VMEM((BLK_PX,), jnp.float32),
        pltpu.VMEM((BLK_PX,), jnp.float32),
        pltpu.VMEM((BLK_PX,), jnp.float32),
        pltpu.SemaphoreType.DMA,
        pltpu.SemaphoreType.DMA,
        pltpu.SemaphoreType.DMA,
        pltpu.SemaphoreType.DMA,
    ],
)(_affine_body)


@jax.jit
def kernel(x, transform):
    x2 = x.reshape(NPLANES * NPIX)
    ainv = jnp.linalg.inv(transform[:, :4].reshape(B, 2, 2))
    params = jnp.concatenate([ainv.reshape(B, 4), transform[:, 4:6]], axis=1)
    params = jnp.pad(params, ((0, 0), (0, NLANES - 6)))
    out = _affine_sc(x2, params)
    return out.reshape(x.shape)
